# baseline (speedup 1.0000x reference)
import functools

import jax
import jax.numpy as jnp
from jax import lax
from jax.experimental import pallas as pl
from jax.experimental.pallas import tpu as pltpu
from jax.experimental.pallas import tpu_sc as plsc

_VOCAB = 1000000
_EMB = 64
_B = 4096
_L = 200
_SCALE = 8.0

_NC = 2
_NS = 16
_NW = _NC * _NS
_BPW = _B // _NW
_TPW = _BPW * _L
_DEPTH = 2


def _emb_body(tokens_hbm, table_hbm, out_hbm, tok_v, idxt, gbuf, obuf, *sems):
    gsems = sems[:_DEPTH]
    osems = sems[_DEPTH:]

    wid = lax.axis_index("s") * _NC + lax.axis_index("c")

    pltpu.sync_copy(tokens_hbm.at[pl.ds(wid * _TPW, _TPW)], tok_v)

    iota = lax.iota(jnp.int32, 16)
    i200 = iota * _L

    def tr_l(l, c):
        for j in range(_BPW // 16):
            t = plsc.load_gather(tok_v, [i200 + (j * 16 * _L + l)])
            idxt[l, pl.ds(j * 16, 16)] = lax.shift_right_logical(t, 1)
        return c

    lax.fori_loop(0, _L, tr_l, 0)

    def start_gather(l, k):
        pltpu.async_copy(table_hbm.at[idxt.at[l]], gbuf.at[k], gsems[k])

    def wait_gather(l, k):
        pltpu.make_async_copy(
            table_hbm.at[idxt.at[l]], gbuf.at[k], gsems[k]
        ).wait()

    def start_out(l, k):
        for cr in range(_EMB // 8):
            pltpu.async_copy(
                obuf.at[k, pl.ds(cr * 8, 8)],
                out_hbm.at[l, cr, wid],
                osems[k],
            )

    def wait_out(l, k):
        for cr in range(_EMB // 8):
            pltpu.make_async_copy(
                obuf.at[k, pl.ds(cr * 8, 8)],
                out_hbm.at[l, cr, wid],
                osems[k],
            ).wait()

    for k in range(_DEPTH):
        start_gather(k, k)

    def round_body(i, carry):
        for k in range(_DEPTH):
            l = _DEPTH * i + k
            wait_gather(l, k)

            @pl.when(l >= _DEPTH)
            def _():
                wait_out(l - _DEPTH, k)

            for j in range(_BPW // 16):
                rows_j = iota + j * 16
                t = plsc.load_gather(tok_v, [i200 + (j * 16 * _L + l)])
                par_j = lax.shift_left(jnp.bitwise_and(t, 1), 6)

                def chan(c, cc):
                    v = plsc.load_gather(gbuf.at[k], [rows_j, par_j + c])
                    obuf[k, c, pl.ds(j * 16, 16)] = v * _SCALE
                    return cc

                lax.fori_loop(0, _EMB, chan, 0, unroll=8)

            start_out(l, k)

            @pl.when(l + _DEPTH < _L)
            def _():
                start_gather(l + _DEPTH, k)

        return carry

    lax.fori_loop(0, _L // _DEPTH, round_body, 0)

    for k in range(_DEPTH):
        wait_out(_L - _DEPTH + k, k)


@jax.jit
def _embed(tokens, table):
    run = functools.partial(
        pl.kernel,
        mesh=plsc.VectorSubcoreMesh(core_axis_name="c", subcore_axis_name="s"),
        out_type=jax.ShapeDtypeStruct((_L, _EMB // 8, _NW, 8, 128), jnp.float32),
        scratch_types=[
            pltpu.VMEM((_TPW,), jnp.int32),
            pltpu.VMEM((_L, _BPW), jnp.int32),
            pltpu.VMEM((_DEPTH, _BPW, 128), jnp.float32),
            pltpu.VMEM((_DEPTH, _EMB, _BPW), jnp.float32),
        ]
        + [pltpu.SemaphoreType.DMA] * (2 * _DEPTH),
        compiler_params=pltpu.CompilerParams(needs_layout_passes=False),
    )(_emb_body)
    return run(tokens.reshape(_B * _L), table.reshape(_VOCAB // 2, 2 * _EMB))


def kernel(tokens, table):
    th = _embed(tokens, table)
    return th.transpose((2, 4, 0, 1, 3)).reshape(_B, _L, _EMB)


# speedup vs baseline: 1.5515x; 1.5515x over previous
import functools

import jax
import jax.numpy as jnp
from jax import lax
from jax.experimental import pallas as pl
from jax.experimental.pallas import tpu as pltpu
from jax.experimental.pallas import tpu_sc as plsc

_VOCAB = 1000000
_EMB = 64
_B = 4096
_L = 200
_N = _B * _L
_SCALE = 8.0

_NC = 2
_NS = 16
_NW = _NC * _NS
_CH = 128
_GRP = _N // (_NW * _CH)


def _emb_body(tokens_hbm, table_hbm, out_hbm, idx_v, rows_v, gsem0, gsem1):
    wid = lax.axis_index("s") * _NC + lax.axis_index("c")
    g0 = wid * _GRP

    pltpu.sync_copy(tokens_hbm.at[pl.ds(g0, _GRP)], idx_v)

    gsems = (gsem0, gsem1)

    def start_gather(g, b):
        pltpu.async_copy(table_hbm.at[idx_v.at[g]], rows_v.at[b], gsems[b])

    def wait_gather(g, b):
        pltpu.make_async_copy(
            table_hbm.at[idx_v.at[g]], rows_v.at[b], gsems[b]
        ).wait()

    start_gather(0, 0)
    start_gather(1, 1)

    def pair_body(i, carry):
        for b in range(2):
            g = 2 * i + b
            wait_gather(g, b)

            def scale_row(r, c):
                for j in range(_EMB // 16):
                    sl = pl.ds(j * 16, 16)
                    rows_v[b, r, sl] = rows_v[b, r, sl] * _SCALE
                return c

            lax.fori_loop(0, _CH, scale_row, 0, unroll=4)

            pltpu.sync_copy(
                rows_v.at[b], out_hbm.at[pl.ds((g0 + g) * _CH, _CH)]
            )

            @pl.when(g + 2 < _GRP)
            def _():
                start_gather(g + 2, b)

        return carry

    lax.fori_loop(0, _GRP // 2, pair_body, 0)


@jax.jit
def _embed(tokens2d, table):
    run = functools.partial(
        pl.kernel,
        mesh=plsc.VectorSubcoreMesh(core_axis_name="c", subcore_axis_name="s"),
        out_type=jax.ShapeDtypeStruct((_N, _EMB), jnp.float32),
        scratch_types=[
            pltpu.VMEM((_GRP, _CH), jnp.int32),
            pltpu.VMEM((2, _CH, _EMB), jnp.float32),
            pltpu.SemaphoreType.DMA,
            pltpu.SemaphoreType.DMA,
        ],
        compiler_params=pltpu.CompilerParams(use_tc_tiling_on_sc=False),
    )(_emb_body)
    return run(tokens2d, table)


def kernel(tokens, table):
    tokens2d = tokens.reshape(_N // _CH, _CH)
    out = _embed(tokens2d, table)
    return out.reshape(_B, _L, _EMB)
